# Initial kernel scaffold; baseline (speedup 1.0000x reference)
import jax
import jax.numpy as jnp
from jax.experimental import pallas as pl


def kernel(node_features, adj_matrix, W, b):
    raise NotImplementedError("write your pallas kernel here")


# speedup vs baseline: 858.8406x; 858.8406x over previous
import jax
import jax.numpy as jnp
from jax.experimental import pallas as pl


def _gcn_body(x_ref, a_ref, w_ref, b_ref, o_ref):
    x = x_ref[...]
    a = a_ref[...].astype(jnp.float32)
    w = w_ref[...]
    b = b_ref[...]
    h = jax.lax.dot_general(
        x, w, (((1,), (1,)), ((), ())), preferred_element_type=jnp.float32
    ) + b
    agg = jax.lax.dot_general(
        a, h, (((1,), (1,)), ((), ())), preferred_element_type=jnp.float32
    )
    deg = jnp.sum(a, axis=1)
    o_ref[...] = (agg + h) / deg[None, :]


def kernel(node_features, adj_matrix, W, b):
    x = node_features[0]
    a = adj_matrix[0].astype(jnp.int32)
    out = pl.pallas_call(
        _gcn_body,
        out_shape=jax.ShapeDtypeStruct(x.shape, jnp.float32),
    )(x, a, W, b.reshape(1, -1))
    return out[None]
